# baseline (speedup 1.0000x reference)
import functools

import jax
import jax.numpy as jnp
from jax import lax
from jax.experimental import pallas as pl
from jax.experimental.pallas import tpu as pltpu

BATCH = 16384
EMBED_K = 64
CH = 512
NSTEP = BATCH // CH


def _tc_body(users_smem, items_smem, gu_any, gi_any,
             xui_ref, gu_out, gi_out, rows_u, rows_i, sem_u, sem_i):
    step = pl.program_id(0)
    cbase = step * CH

    def issue(j, _):
        r_u = users_smem[cbase + j]
        pltpu.make_async_copy(gu_any.at[r_u], rows_u.at[j], sem_u).start()
        r_i = items_smem[cbase + j]
        pltpu.make_async_copy(gi_any.at[r_i], rows_i.at[j], sem_i).start()
        return 0

    lax.fori_loop(0, CH, issue, 0, unroll=16)
    pltpu.make_async_copy(gu_any.at[pl.ds(0, CH)], rows_u, sem_u).wait()
    pltpu.make_async_copy(gi_any.at[pl.ds(0, CH)], rows_i, sem_i).wait()

    u = rows_u[...]
    v = rows_i[...]
    gu_out[...] = u
    gi_out[...] = v
    xui_ref[...] = jnp.sum(u * v, axis=1)


_call = pl.pallas_call(
    _tc_body,
    grid_spec=pltpu.PrefetchScalarGridSpec(
        num_scalar_prefetch=2,
        grid=(NSTEP,),
        in_specs=[
            pl.BlockSpec(memory_space=pl.ANY),
            pl.BlockSpec(memory_space=pl.ANY),
        ],
        out_specs=[
            pl.BlockSpec((CH,), lambda i, users, items: (i,)),
            pl.BlockSpec((CH, EMBED_K), lambda i, users, items: (i, 0)),
            pl.BlockSpec((CH, EMBED_K), lambda i, users, items: (i, 0)),
        ],
        scratch_shapes=[
            pltpu.VMEM((CH, EMBED_K), jnp.float32),
            pltpu.VMEM((CH, EMBED_K), jnp.float32),
            pltpu.SemaphoreType.DMA,
            pltpu.SemaphoreType.DMA,
        ],
    ),
    out_shape=[
        jax.ShapeDtypeStruct((BATCH,), jnp.float32),
        jax.ShapeDtypeStruct((BATCH, EMBED_K), jnp.float32),
        jax.ShapeDtypeStruct((BATCH, EMBED_K), jnp.float32),
    ],
)


def kernel(users, items, Gu, Gi):
    xui, gamma_u, gamma_i = _call(users, items, Gu, Gi)
    return (xui, gamma_u, gamma_i)
